# baseline (speedup 1.0000x reference)
import functools

import jax
import jax.numpy as jnp
from jax import lax
from jax.experimental import pallas as pl
from jax.experimental.pallas import tpu as pltpu
from jax.experimental.pallas import tpu_sc as plsc

_N = 10000
_L = 16
_G = _N // _L
_U = 5
_GU = _G // _U


def _build():
    mesh = plsc.VectorSubcoreMesh(core_axis_name="c", subcore_axis_name="s")

    @functools.partial(
        pl.kernel,
        mesh=mesh,
        out_type=jax.ShapeDtypeStruct((_N,), jnp.int32),
        compiler_params=pltpu.CompilerParams(needs_layout_passes=False),
        scratch_types=[
            pltpu.VMEM((_N + _L,), jnp.int32),
            pltpu.VMEM((_N,), jnp.int32),
            pltpu.VMEM((_N + _L,), jnp.int32),
            pltpu.VMEM((_N,), jnp.int32),
            pltpu.VMEM((_N,), jnp.int32),
            pltpu.VMEM((_N,), jnp.int32),
            pltpu.VMEM((_N,), jnp.int32),
            pltpu.SemaphoreType.DMA,
        ],
    )
    def k(kl_hbm, vl_hbm, kr_hbm, vr_hbm, ids_hbm, zeros_hbm, out_hbm,
          kl_v, vl_v, kr_v, vr_v, wl_v, wr_v, mask_v, sem):
        cid = lax.axis_index("c")
        sid = lax.axis_index("s")

        @pl.when((cid == 0) & (sid == 0))
        def _():
            hs = [
                pltpu.async_copy(kl_hbm, kl_v.at[pl.ds(0, _N)], sem),
                pltpu.async_copy(vl_hbm, vl_v, sem),
                pltpu.async_copy(kr_hbm, kr_v.at[pl.ds(0, _N)], sem),
                pltpu.async_copy(vr_hbm, vr_v, sem),
                pltpu.async_copy(ids_hbm, wl_v, sem),
                pltpu.async_copy(ids_hbm, wr_v, sem),
                pltpu.async_copy(zeros_hbm, mask_v, sem),
            ]
            for h in hs:
                h.wait()
            lanes = lax.iota(jnp.int32, _L)
            sentinel = jnp.full((_L,), _N, jnp.int32)
            kl_v[pl.ds(_N, _L)] = sentinel
            kr_v[pl.ds(_N, _L)] = sentinel
            mask_v[pl.ds(0, _L)] = jnp.where(lanes == 0, jnp.int32(1), jnp.int32(0))

            def phase_a(g, c):
                for u in range(_U):
                    base = (g * _U + u) * _L
                    k1 = kl_v[pl.ds(base, _L)]
                    keep1 = k1 != kl_v[pl.ds(base + 1, _L)]
                    plsc.store_scatter(wl_v, [k1], vl_v[pl.ds(base, _L)], mask=keep1)
                    k2 = kr_v[pl.ds(base, _L)]
                    keep2 = k2 != kr_v[pl.ds(base + 1, _L)]
                    plsc.store_scatter(wr_v, [k2], vr_v[pl.ds(base, _L)], mask=keep2)
                return c

            lax.fori_loop(0, _GU, phase_a, jnp.int32(0))

            def sweep_body(g, ch):
                for u in range(_U):
                    base = (g * _U + u) * _L
                    cur = mask_v[pl.ds(base, _L)]
                    lv = plsc.load_gather(mask_v, [wl_v[pl.ds(base, _L)]])
                    rv = plsc.load_gather(mask_v, [wr_v[pl.ds(base, _L)]])
                    new = cur | lv | rv
                    mask_v[pl.ds(base, _L)] = new
                    ch = ch | (new ^ cur)
                return ch

            def w_cond(c):
                return c != 0

            def w_body(c):
                chv = lax.fori_loop(0, _GU, sweep_body, jnp.zeros(_L, jnp.int32))
                return jnp.max(chv)

            lax.while_loop(w_cond, w_body, jnp.int32(1))
            pltpu.sync_copy(mask_v, out_hbm)

    return k


_k = _build()


def kernel(thresholds, rules_left, rules_right, binary_ops, left, right):
    iota2 = jnp.broadcast_to(jnp.arange(_N, dtype=jnp.int32), (2, _N))
    keys, vals = lax.sort(
        (jnp.stack([left, right]), iota2),
        num_keys=1, dimension=1, is_stable=False,
    )
    ids = jnp.arange(_N, dtype=jnp.int32)
    zeros = jnp.zeros(_N, jnp.int32)
    out = _k(keys[0], vals[0], keys[1], vals[1], ids, zeros)
    return out != 0
